# baseline (speedup 1.0000x reference)
import functools

import jax
import jax.numpy as jnp
from jax import lax
from jax.experimental import pallas as pl
from jax.experimental.pallas import tpu as pltpu
from jax.experimental.pallas import tpu_sc as plsc


@functools.lru_cache(maxsize=None)
def _make_sc_gather(n_rows: int, emb_dim: int, n_tiles: int):
    info = plsc.get_sparse_core_info()
    nc, ns, lanes = info.num_cores, info.num_subcores, info.num_lanes
    nw = nc * ns
    rows_per_w = n_rows // nw
    chunk = 128
    n_chunk = rows_per_w // chunk
    wide = 2 * emb_dim
    assert rows_per_w % chunk == 0 and n_rows % nw == 0

    mesh = plsc.VectorSubcoreMesh(core_axis_name="c", subcore_axis_name="s")

    @functools.partial(
        pl.kernel,
        mesh=mesh,
        out_type=jax.ShapeDtypeStruct((n_rows, emb_dim), jnp.float32),
        scratch_types=[
            pltpu.VMEM((n_chunk, chunk), jnp.int32),
            pltpu.VMEM((n_chunk, chunk), jnp.int32),
            pltpu.VMEM((chunk, 2 * emb_dim), jnp.float32),
            pltpu.VMEM((chunk, 2 * emb_dim), jnp.float32),
            pltpu.VMEM((rows_per_w, emb_dim), jnp.float32),
            pltpu.SemaphoreType.DMA,
            pltpu.SemaphoreType.DMA,
        ],
        compiler_params=pltpu.CompilerParams(needs_layout_passes=False),
    )
    def gather_k(tidx_hbm, sub_hbm, table_hbm, out_hbm,
                 tidx_v, sub_v, buf_a, buf_b, out_v, sem_a, sem_b):
        wid = lax.axis_index("s") * nc + lax.axis_index("c")
        pltpu.sync_copy(tidx_hbm.at[wid], tidx_v)
        pltpu.sync_copy(sub_hbm.at[wid], sub_v)
        lane_iota = lax.iota(jnp.int32, lanes)
        bufs = [buf_a, buf_b]
        sems = [sem_a, sem_b]

        def issue(k):
            return pltpu.async_copy(table_hbm.at[tidx_v.at[k]],
                                    bufs[k % 2], sems[k % 2])

        def extract(k):
            for g in range(chunk // lanes):
                tok16 = lane_iota + g * lanes
                off16 = sub_v[k, pl.ds(g * lanes, lanes)]
                dst16 = tok16 + k * chunk

                def col_body(ci, _):
                    for u in range(4):
                        c16 = jnp.full((lanes,), ci * 4 + u, jnp.int32)
                        vals = plsc.load_gather(bufs[k % 2],
                                                [tok16, off16 + c16])
                        plsc.store_scatter(out_v, [dst16, c16], vals)
                    return 0

                lax.fori_loop(0, emb_dim // 4, col_body, 0)

        copies = [issue(0)]
        for k in range(n_chunk):
            if k + 1 < n_chunk:
                copies.append(issue(k + 1))
            copies[k].wait()
            extract(k)
        pltpu.sync_copy(out_v, out_hbm.at[pl.ds(wid * rows_per_w, rows_per_w)])

    return gather_k


def _lstm_body(L, H, emb_ref, mask_ref, wih_ref, whh_ref, b_ref,
               h_out, c_out, h_s, c_s):
    t = pl.program_id(0)

    @pl.when(t == 0)
    def _init():
        h_s[...] = jnp.zeros_like(h_s)
        c_s[...] = jnp.zeros_like(c_s)

    xt = emb_ref[0] * mask_ref[0]
    h = h_s[...]
    c = c_s[...]
    gates = lax.dot_general(xt, wih_ref[...], (((1,), (1,)), ((), ())),
                            preferred_element_type=jnp.float32)
    gates = gates + lax.dot_general(h, whh_ref[...], (((1,), (1,)), ((), ())),
                                    preferred_element_type=jnp.float32)
    gates = gates + b_ref[...]
    i = jax.nn.sigmoid(gates[:, 0:H])
    f = jax.nn.sigmoid(gates[:, H:2 * H])
    g = jnp.tanh(gates[:, 2 * H:3 * H])
    o = jax.nn.sigmoid(gates[:, 3 * H:4 * H])
    c_new = f * c + i * g
    h_new = o * jnp.tanh(c_new)
    h_s[...] = h_new
    c_s[...] = c_new

    @pl.when(t == L - 1)
    def _emit():
        h_out[...] = h_new
        c_out[...] = c_new


def _lstm(embT, mask3, W_ih, W_hh, b2):
    L, B, E = embT.shape
    H = W_hh.shape[1]
    return pl.pallas_call(
        functools.partial(_lstm_body, L, H),
        grid=(L,),
        in_specs=[
            pl.BlockSpec((1, B, E), lambda t: (t, 0, 0)),
            pl.BlockSpec((1, B, 1), lambda t: (t, 0, 0)),
            pl.BlockSpec((4 * H, E), lambda t: (0, 0)),
            pl.BlockSpec((4 * H, H), lambda t: (0, 0)),
            pl.BlockSpec((1, 4 * H), lambda t: (0, 0)),
        ],
        out_specs=[
            pl.BlockSpec((B, H), lambda t: (0, 0)),
            pl.BlockSpec((B, H), lambda t: (0, 0)),
        ],
        out_shape=[jax.ShapeDtypeStruct((B, H), jnp.float32)] * 2,
        scratch_shapes=[
            pltpu.VMEM((B, H), jnp.float32),
            pltpu.VMEM((B, H), jnp.float32),
        ],
    )(embT, mask3, W_ih, W_hh, b2)


def kernel(x, table, W_ih, W_hh, b_ih, b_hh):
    B, L = x.shape
    V, E = table.shape
    H = W_hh.shape[1]
    nw, chunk = 32, 128

    xT = jnp.transpose(x)
    flat_idx = xT.reshape(-1)
    tidx = (flat_idx >> 1).reshape(nw, -1, chunk)
    sub = ((flat_idx & 1) * E).reshape(nw, -1, chunk)
    table3 = table.reshape(V // 2, 2 * E)

    emb_flat = _make_sc_gather(L * B, E, V // 8)(tidx, sub, table3)
    embT = emb_flat.reshape(L, B, E)
    mask3 = (xT != 0).astype(jnp.float32).reshape(L, B, 1)
    b2 = (b_ih + b_hh).reshape(1, 4 * H)

    hN, cN = _lstm(embT, mask3, W_ih, W_hh, b2)
    return hN[None, :, :], cN[None, :, :]


# speedup vs baseline: 2.1867x; 2.1867x over previous
import functools

import jax
import jax.numpy as jnp
from jax import lax
from jax.experimental import pallas as pl
from jax.experimental.pallas import tpu as pltpu
from jax.experimental.pallas import tpu_sc as plsc


_NBUF = 4


@functools.lru_cache(maxsize=None)
def _make_sc_gather(n_rows: int, emb_dim: int, n_tiles: int):
    info = plsc.get_sparse_core_info()
    nc, ns, lanes = info.num_cores, info.num_subcores, info.num_lanes
    nw = nc * ns
    rows_per_w = n_rows // nw
    n_groups = rows_per_w // lanes
    assert rows_per_w % lanes == 0 and n_rows % nw == 0
    assert n_groups % _NBUF == 0

    mesh = plsc.VectorSubcoreMesh(core_axis_name="c", subcore_axis_name="s")

    @functools.partial(
        pl.kernel,
        mesh=mesh,
        out_type=jax.ShapeDtypeStruct((n_rows, emb_dim), jnp.float32),
        scratch_types=[
            pltpu.VMEM((8, 128), jnp.int32),
            pltpu.VMEM((8, 128), jnp.int32),
            [pltpu.VMEM((lanes, 8, emb_dim), jnp.float32)] * _NBUF,
            pltpu.VMEM((8 * lanes, emb_dim), jnp.float32),
            [pltpu.SemaphoreType.DMA] * _NBUF,
        ],
        compiler_params=pltpu.CompilerParams(needs_layout_passes=False),
    )
    def gather_k(tidx_hbm, sub_hbm, table_hbm, out_hbm,
                 tidx_v, sub_v, bufs, out_v, sems):
        wid = lax.axis_index("s") * nc + lax.axis_index("c")
        pltpu.sync_copy(tidx_hbm.at[wid], tidx_v)
        pltpu.sync_copy(sub_hbm.at[wid], sub_v)
        lane_iota = lax.iota(jnp.int32, lanes)
        lane_masks = [(lane_iota == j).astype(jnp.int32) for j in range(lanes)]

        def idx16(ref, g):
            r16 = jnp.full((lanes,), g >> 3, jnp.int32)
            c16 = lane_iota + ((g & 7) * lanes)
            return plsc.load_gather(ref, [r16, c16])

        def issue(g, q):
            t16 = idx16(tidx_v, g)
            for j in range(lanes):
                t_s = jnp.sum(t16 * lane_masks[j])
                pltpu.async_copy(table_hbm.at[pl.ds(t_s, 1)],
                                 bufs[q].at[pl.ds(j, 1)], sems[q])

        def drain(q):
            pltpu.make_async_copy(table_hbm.at[pl.ds(0, lanes)],
                                  bufs[q], sems[q]).wait()

        def extract(g, q):
            m16 = idx16(sub_v, g)
            dst16 = lane_iota + (g & 7) * lanes

            def col_body(ci, _):
                for u in range(4):
                    c16 = jnp.full((lanes,), ci * 4 + u, jnp.int32)
                    vals = plsc.load_gather(bufs[q], [lane_iota, m16, c16])
                    plsc.store_scatter(out_v, [dst16, c16], vals)
                return 0

            lax.fori_loop(0, emb_dim // 4, col_body, 0)

        for q in range(_NBUF - 1):
            issue(q, q)

        flush_toks = 8 * lanes

        def quad_body(p, _):
            g0 = p * _NBUF
            for q in range(_NBUF):
                g = g0 + q

                @pl.when(g + _NBUF - 1 < n_groups)
                def _issue_ahead():
                    issue(g + _NBUF - 1, (q + _NBUF - 1) % _NBUF)

                drain(q)
                extract(g, q)

            @pl.when(p % 2 == 1)
            def _flush():
                pltpu.sync_copy(
                    out_v,
                    out_hbm.at[pl.ds(wid * rows_per_w + (p // 2) * flush_toks,
                                     flush_toks)])
            return 0

        lax.fori_loop(0, n_groups // _NBUF, quad_body, 0)

    return gather_k


def _lstm_body(L, H, emb_ref, mask_ref, wih_ref, whh_ref, b_ref,
               h_out, c_out, h_s, c_s):
    t = pl.program_id(0)

    @pl.when(t == 0)
    def _init():
        h_s[...] = jnp.zeros_like(h_s)
        c_s[...] = jnp.zeros_like(c_s)

    xt = emb_ref[0] * mask_ref[0]
    h = h_s[...]
    c = c_s[...]
    gates = lax.dot_general(xt, wih_ref[...], (((1,), (1,)), ((), ())),
                            preferred_element_type=jnp.float32)
    gates = gates + lax.dot_general(h, whh_ref[...], (((1,), (1,)), ((), ())),
                                    preferred_element_type=jnp.float32)
    gates = gates + b_ref[...]
    i = jax.nn.sigmoid(gates[:, 0:H])
    f = jax.nn.sigmoid(gates[:, H:2 * H])
    g = jnp.tanh(gates[:, 2 * H:3 * H])
    o = jax.nn.sigmoid(gates[:, 3 * H:4 * H])
    c_new = f * c + i * g
    h_new = o * jnp.tanh(c_new)
    h_s[...] = h_new
    c_s[...] = c_new

    @pl.when(t == L - 1)
    def _emit():
        h_out[...] = h_new
        c_out[...] = c_new


def _lstm(embT, mask3, W_ih, W_hh, b2):
    L, B, E = embT.shape
    H = W_hh.shape[1]
    return pl.pallas_call(
        functools.partial(_lstm_body, L, H),
        grid=(L,),
        in_specs=[
            pl.BlockSpec((1, B, E), lambda t: (t, 0, 0)),
            pl.BlockSpec((1, B, 1), lambda t: (t, 0, 0)),
            pl.BlockSpec((4 * H, E), lambda t: (0, 0)),
            pl.BlockSpec((4 * H, H), lambda t: (0, 0)),
            pl.BlockSpec((1, 4 * H), lambda t: (0, 0)),
        ],
        out_specs=[
            pl.BlockSpec((B, H), lambda t: (0, 0)),
            pl.BlockSpec((B, H), lambda t: (0, 0)),
        ],
        out_shape=[jax.ShapeDtypeStruct((B, H), jnp.float32)] * 2,
        scratch_shapes=[
            pltpu.VMEM((B, H), jnp.float32),
            pltpu.VMEM((B, H), jnp.float32),
        ],
    )(embT, mask3, W_ih, W_hh, b2)


def kernel(x, table, W_ih, W_hh, b_ih, b_hh):
    B, L = x.shape
    V, E = table.shape
    H = W_hh.shape[1]
    nw, chunk = 32, 128

    xT = jnp.transpose(x)
    flat_idx = xT.reshape(-1)
    tidx = (flat_idx >> 3).reshape(nw, -1, chunk)
    sub = (flat_idx & 7).reshape(nw, -1, chunk)
    pad_rows = 8 - tidx.shape[1]
    tidx = jnp.pad(tidx, ((0, 0), (0, pad_rows), (0, 0)))
    sub = jnp.pad(sub, ((0, 0), (0, pad_rows), (0, 0)))
    table3 = table.reshape(V // 8, 8, E)

    emb_flat = _make_sc_gather(L * B, E, V // 8)(tidx, sub, table3)
    embT = emb_flat.reshape(L, B, E)
    mask3 = (xT != 0).astype(jnp.float32).reshape(L, B, 1)
    b2 = (b_ih + b_hh).reshape(1, 4 * H)

    hN, cN = _lstm(embT, mask3, W_ih, W_hh, b2)
    return hN[None, :, :], cN[None, :, :]
